# baseline (speedup 1.0000x reference)
import functools

import jax
import jax.numpy as jnp
from jax import lax
from jax.experimental import pallas as pl
from jax.experimental.pallas import tpu as pltpu
from jax.experimental.pallas import tpu_sc as plsc

N = 10000
E = 160000
D = 256
H = 256
HH = 128
NPAD = 10240

NC = 2
NS = 16
CH = 64
SB = 2
NCH = 160
NCH2 = NCH // SB
EPT = NCH * CH

NH = 5120
NDMP = 128
NACC = NH + NDMP
RPT = NH // NS
ZPT = NACC // NS

BLK = 80
NBLK = N // BLK
PBLK = NPAD // BLK


def _chunks(total, step=128):
    out, o = [], 0
    while o < total:
        out.append((o, min(step, total - o)))
        o += min(step, total - o)
    return out


def _sc_agg_body(pk_hbm, hstack_hbm, out_hbm,
                 pk_v, idx_v, dst_v, rows_v, rows2_v, stage_v, agg_sh,
                 sem, sem2):
    c = lax.axis_index("c")
    s = lax.axis_index("s")

    off = c * NPAD
    rot = lax.iota(jnp.int32, 16)

    def pass_body(p, carry):
        lo = p * NH

        def zrow(r, carry):
            for k in range(HH // 16):
                stage_v[r, pl.ds(k * 16, 16)] = jnp.zeros((16,), jnp.float32)
            return carry
        lax.fori_loop(0, 128, zrow, 0)
        for (o, sz) in _chunks(ZPT):
            pltpu.sync_copy(stage_v.at[pl.ds(0, sz)],
                            agg_sh.at[pl.ds(s * ZPT + o, sz)])
        plsc.subcore_barrier()

        def sb_body(sb, carry):
            pltpu.sync_copy(pk_hbm.at[s, sb], pk_v)

            def urow(r, carry):
                for k in range(CH // 16):
                    g = r * (CH // 16) + k
                    sl = pl.ds(k * 16, 16)
                    v = pk_v[r, sl]
                    idx_v[r, sl] = (v & 0xFFFF) + off
                    d = lax.shift_right_logical(v, 16) - lo
                    dump = NH + (g % 8) * 16 + rot
                    inr = (d >= 0) & (d < NH)
                    dst_v[r, sl] = jnp.where(inr, d, dump)
                return carry
            lax.fori_loop(0, NCH2, urow, 0)

            pltpu.async_copy(hstack_hbm.at[idx_v.at[0]], rows_v, sem)

            def obody(jj, carry):
                j0 = 2 * jj
                j1 = j0 + 1
                jn = jnp.minimum(j0 + 2, NCH2 - 1)
                pltpu.make_async_copy(
                    hstack_hbm.at[idx_v.at[j0]], rows_v, sem).wait()
                pltpu.async_copy(hstack_hbm.at[idx_v.at[j1]], rows2_v, sem2)
                pltpu.sync_copy(rows_v, agg_sh.at[dst_v.at[j0]], add=True)
                pltpu.make_async_copy(
                    hstack_hbm.at[idx_v.at[j1]], rows2_v, sem2).wait()
                pltpu.async_copy(hstack_hbm.at[idx_v.at[jn]], rows_v, sem)
                pltpu.sync_copy(rows2_v, agg_sh.at[dst_v.at[j1]], add=True)
                return carry
            lax.fori_loop(0, NCH2 // 2, obody, 0)
            pltpu.make_async_copy(
                hstack_hbm.at[idx_v.at[NCH2 - 1]], rows_v, sem).wait()
            return carry
        lax.fori_loop(0, SB, sb_body, 0)
        plsc.subcore_barrier()

        out_base = c * NPAD + lo + s * RPT
        for (o, sz) in _chunks(RPT):
            pltpu.sync_copy(agg_sh.at[pl.ds(s * RPT + o, sz)],
                            stage_v.at[pl.ds(0, sz)])
            pltpu.sync_copy(stage_v.at[pl.ds(0, sz)],
                            out_hbm.at[pl.ds(out_base + o, sz)])
        plsc.subcore_barrier()
        return carry

    lax.fori_loop(0, 2, pass_body, 0)


@functools.cache
def _make_sc_agg():
    return pl.kernel(
        _sc_agg_body,
        out_type=jax.ShapeDtypeStruct((2 * NPAD, HH), jnp.float32),
        mesh=plsc.VectorSubcoreMesh(core_axis_name="c", subcore_axis_name="s",
                                    num_cores=NC, num_subcores=NS),
        scratch_types=[
            pltpu.VMEM((NCH2, CH), jnp.int32),
            pltpu.VMEM((NCH2, CH), jnp.int32),
            pltpu.VMEM((NCH2, CH), jnp.int32),
            pltpu.VMEM((CH, HH), jnp.float32),
            pltpu.VMEM((CH, HH), jnp.float32),
            pltpu.VMEM((128, HH), jnp.float32),
            pltpu.VMEM_SHARED((NACC, HH), jnp.float32),
            pltpu.SemaphoreType.DMA,
            pltpu.SemaphoreType.DMA,
        ],
    )


def _sc_agg(pk_rs, hstack):
    return _make_sc_agg()(pk_rs, hstack)


def _l0_body(x_ref, w_ref, b_ref, o_ref):
    o_ref[...] = jnp.maximum(
        jnp.dot(x_ref[...], w_ref[...], preferred_element_type=jnp.float32)
        + b_ref[0], 0.0)


def _tc_layer0(x, W0, b0):
    return pl.pallas_call(
        _l0_body,
        grid=(NC, NBLK),
        in_specs=[
            pl.BlockSpec((BLK, D), lambda c, i: (i, 0)),
            pl.BlockSpec((D, HH), lambda c, i: (0, c)),
            pl.BlockSpec((1, 1, HH), lambda c, i: (c, 0, 0)),
        ],
        out_specs=pl.BlockSpec((BLK, HH), lambda c, i: (c * PBLK + i, 0)),
        out_shape=jax.ShapeDtypeStruct((2 * NPAD, HH), jnp.float32),
    )(x, W0, b0.reshape(NC, 1, HH))


def _lmid_body(a_ref, b2_ref, w_ref, b_ref, o_ref):
    acc = jnp.dot(a_ref[...], w_ref[:HH, :], preferred_element_type=jnp.float32)
    acc += jnp.dot(b2_ref[...], w_ref[HH:, :], preferred_element_type=jnp.float32)
    o_ref[...] = jnp.maximum(acc + b_ref[0], 0.0)


def _tc_layer(agg, W, b):
    return pl.pallas_call(
        _lmid_body,
        grid=(NC, NBLK),
        in_specs=[
            pl.BlockSpec((BLK, HH), lambda c, i: (i, 0)),
            pl.BlockSpec((BLK, HH), lambda c, i: (PBLK + i, 0)),
            pl.BlockSpec((H, HH), lambda c, i: (0, c)),
            pl.BlockSpec((1, 1, HH), lambda c, i: (c, 0, 0)),
        ],
        out_specs=pl.BlockSpec((BLK, HH), lambda c, i: (c * PBLK + i, 0)),
        out_shape=jax.ShapeDtypeStruct((2 * NPAD, HH), jnp.float32),
    )(agg, agg, W, b.reshape(NC, 1, HH))


def _final_body(a_ref, b2_ref, w_ref, b_ref, wp_ref, bp_ref, o_ref):
    acc = jnp.dot(a_ref[...], w_ref[:HH, :], preferred_element_type=jnp.float32)
    acc += jnp.dot(b2_ref[...], w_ref[HH:, :], preferred_element_type=jnp.float32)
    h = jnp.maximum(acc + b_ref[...], 0.0)
    o_ref[...] = jnp.dot(h, wp_ref[...], preferred_element_type=jnp.float32) \
        + bp_ref[...]


def _tc_final(agg, W, b, Wp, bp):
    return pl.pallas_call(
        _final_body,
        grid=(NBLK,),
        in_specs=[
            pl.BlockSpec((BLK, HH), lambda i: (i, 0)),
            pl.BlockSpec((BLK, HH), lambda i: (PBLK + i, 0)),
            pl.BlockSpec((H, H), lambda i: (0, 0)),
            pl.BlockSpec((1, H), lambda i: (0, 0)),
            pl.BlockSpec((H, 1), lambda i: (0, 0)),
            pl.BlockSpec((1, 1), lambda i: (0, 0)),
        ],
        out_specs=pl.BlockSpec((BLK, 1), lambda i: (i, 0)),
        out_shape=jax.ShapeDtypeStruct((N, 1), jnp.float32),
    )(agg, agg, W, b.reshape(1, H), Wp, bp.reshape(1, 1))


def kernel(x, edge_index, W0, b0, W1, b1, W2, b2, W3, b3, W4, b4, Wp, bp):
    src = edge_index[0]
    dst = edge_index[1]
    pad = EPT - E // NS
    pk = (dst << 16) | src
    pk_rs = jnp.pad(pk.reshape(NS, E // NS), ((0, 0), (0, pad)),
                    constant_values=N << 16).reshape(NS, SB, NCH2, CH)

    h = _tc_layer0(x, W0, b0)
    for W, b in ((W1, b1), (W2, b2), (W3, b3)):
        agg = _sc_agg(pk_rs, h)
        h = _tc_layer(agg, W, b)
    agg = _sc_agg(pk_rs, h)
    return _tc_final(agg, W4, b4, Wp, bp)


# speedup vs baseline: 1.6221x; 1.6221x over previous
import functools

import jax
import jax.numpy as jnp
from jax import lax
from jax.experimental import pallas as pl
from jax.experimental.pallas import tpu as pltpu
from jax.experimental.pallas import tpu_sc as plsc

N = 10000
E = 160000
D = 256
H = 256
HH = 128
NPAD = 10240

NC = 2
NS = 16
CH = 64
SB = 4
NCH = 160
NCH2 = NCH // SB
EPT = NCH * CH

NACC = NPAD
RPT = NPAD // NS
ZPT = NACC // NS

BLK = 80
NBLK = N // BLK
PBLK = NPAD // BLK


def _chunks(total, step=128):
    out, o = [], 0
    while o < total:
        out.append((o, min(step, total - o)))
        o += min(step, total - o)
    return out


def _sc_agg_body(pk_hbm, hstack_hbm, out_hbm,
                 pk_v, idx_v, dst_v, rows_v, stage_v, agg_sh, sem):
    c = lax.axis_index("c")
    s = lax.axis_index("s")

    off = c * NPAD

    def zrow(r, carry):
        for k in range(HH // 16):
            stage_v[r, pl.ds(k * 16, 16)] = jnp.zeros((16,), jnp.float32)
        return carry
    lax.fori_loop(0, 128, zrow, 0)
    for (o, sz) in _chunks(ZPT):
        pltpu.sync_copy(stage_v.at[pl.ds(0, sz)],
                        agg_sh.at[pl.ds(s * ZPT + o, sz)])
    plsc.subcore_barrier()

    def sb_body(sb, carry):
        pltpu.sync_copy(pk_hbm.at[s, sb], pk_v)

        def urow(r, carry):
            for k in range(CH // 16):
                sl = pl.ds(k * 16, 16)
                v = pk_v[r, sl]
                idx_v[r, sl] = (v & 0xFFFF) + off
                dst_v[r, sl] = lax.shift_right_logical(v, 16)
            return carry
        lax.fori_loop(0, NCH2, urow, 0)

        def ebody(j, carry):
            pltpu.async_copy(hstack_hbm.at[idx_v.at[j]], rows_v, sem).wait()
            pltpu.sync_copy(rows_v, agg_sh.at[dst_v.at[j]], add=True)
            return carry
        lax.fori_loop(0, NCH2, ebody, 0)
        return carry
    lax.fori_loop(0, SB, sb_body, 0)
    plsc.subcore_barrier()

    out_base = c * NPAD + s * RPT
    for (o, sz) in _chunks(RPT):
        pltpu.sync_copy(agg_sh.at[pl.ds(s * RPT + o, sz)],
                        stage_v.at[pl.ds(0, sz)])
        pltpu.sync_copy(stage_v.at[pl.ds(0, sz)],
                        out_hbm.at[pl.ds(out_base + o, sz)])


@functools.cache
def _make_sc_agg():
    return pl.kernel(
        _sc_agg_body,
        out_type=jax.ShapeDtypeStruct((2 * NPAD, HH), jnp.float32),
        mesh=plsc.VectorSubcoreMesh(core_axis_name="c", subcore_axis_name="s",
                                    num_cores=NC, num_subcores=NS),
        scratch_types=[
            pltpu.VMEM((NCH2, CH), jnp.int32),
            pltpu.VMEM((NCH2, CH), jnp.int32),
            pltpu.VMEM((NCH2, CH), jnp.int32),
            pltpu.VMEM((CH, HH), jnp.float32),
            pltpu.VMEM((128, HH), jnp.float32),
            pltpu.VMEM_SHARED((NACC, HH), jnp.float32),
            pltpu.SemaphoreType.DMA,
        ],
    )


def _sc_agg(pk_rs, hstack):
    return _make_sc_agg()(pk_rs, hstack)


def _l0_body(x_ref, w_ref, b_ref, o_ref):
    o_ref[...] = jnp.maximum(
        jnp.dot(x_ref[...], w_ref[...], preferred_element_type=jnp.float32)
        + b_ref[0], 0.0)


def _tc_layer0(x, W0, b0):
    return pl.pallas_call(
        _l0_body,
        grid=(NC, NBLK),
        in_specs=[
            pl.BlockSpec((BLK, D), lambda c, i: (i, 0)),
            pl.BlockSpec((D, HH), lambda c, i: (0, c)),
            pl.BlockSpec((1, 1, HH), lambda c, i: (c, 0, 0)),
        ],
        out_specs=pl.BlockSpec((BLK, HH), lambda c, i: (c * PBLK + i, 0)),
        out_shape=jax.ShapeDtypeStruct((2 * NPAD, HH), jnp.float32),
    )(x, W0, b0.reshape(NC, 1, HH))


def _lmid_body(a_ref, b2_ref, w_ref, b_ref, o_ref):
    acc = jnp.dot(a_ref[...], w_ref[:HH, :], preferred_element_type=jnp.float32)
    acc += jnp.dot(b2_ref[...], w_ref[HH:, :], preferred_element_type=jnp.float32)
    o_ref[...] = jnp.maximum(acc + b_ref[0], 0.0)


def _tc_layer(agg, W, b):
    return pl.pallas_call(
        _lmid_body,
        grid=(NC, NBLK),
        in_specs=[
            pl.BlockSpec((BLK, HH), lambda c, i: (i, 0)),
            pl.BlockSpec((BLK, HH), lambda c, i: (PBLK + i, 0)),
            pl.BlockSpec((H, HH), lambda c, i: (0, c)),
            pl.BlockSpec((1, 1, HH), lambda c, i: (c, 0, 0)),
        ],
        out_specs=pl.BlockSpec((BLK, HH), lambda c, i: (c * PBLK + i, 0)),
        out_shape=jax.ShapeDtypeStruct((2 * NPAD, HH), jnp.float32),
    )(agg, agg, W, b.reshape(NC, 1, HH))


def _final_body(a_ref, b2_ref, w_ref, b_ref, wp_ref, bp_ref, o_ref):
    acc = jnp.dot(a_ref[...], w_ref[:HH, :], preferred_element_type=jnp.float32)
    acc += jnp.dot(b2_ref[...], w_ref[HH:, :], preferred_element_type=jnp.float32)
    h = jnp.maximum(acc + b_ref[...], 0.0)
    o_ref[...] = jnp.dot(h, wp_ref[...], preferred_element_type=jnp.float32) \
        + bp_ref[...]


def _tc_final(agg, W, b, Wp, bp):
    return pl.pallas_call(
        _final_body,
        grid=(NBLK,),
        in_specs=[
            pl.BlockSpec((BLK, HH), lambda i: (i, 0)),
            pl.BlockSpec((BLK, HH), lambda i: (PBLK + i, 0)),
            pl.BlockSpec((H, H), lambda i: (0, 0)),
            pl.BlockSpec((1, H), lambda i: (0, 0)),
            pl.BlockSpec((H, 1), lambda i: (0, 0)),
            pl.BlockSpec((1, 1), lambda i: (0, 0)),
        ],
        out_specs=pl.BlockSpec((BLK, 1), lambda i: (i, 0)),
        out_shape=jax.ShapeDtypeStruct((N, 1), jnp.float32),
    )(agg, agg, W, b.reshape(1, H), Wp, bp.reshape(1, 1))


def kernel(x, edge_index, W0, b0, W1, b1, W2, b2, W3, b3, W4, b4, Wp, bp):
    src = edge_index[0]
    dst = edge_index[1]
    pad = EPT - E // NS
    pk = (dst << 16) | src
    pk_rs = jnp.pad(pk.reshape(NS, E // NS), ((0, 0), (0, pad)),
                    constant_values=N << 16).reshape(NS, SB, NCH2, CH)

    h = _tc_layer0(x, W0, b0)
    for W, b in ((W1, b1), (W2, b2), (W3, b3)):
        agg = _sc_agg(pk_rs, h)
        h = _tc_layer(agg, W, b)
    agg = _sc_agg(pk_rs, h)
    return _tc_final(agg, W4, b4, Wp, bp)


# speedup vs baseline: 2.1600x; 1.3316x over previous
import functools

import jax
import jax.numpy as jnp
from jax import lax
from jax.experimental import pallas as pl
from jax.experimental.pallas import tpu as pltpu
from jax.experimental.pallas import tpu_sc as plsc

N = 10000
E = 160000
D = 256
H = 256
HH = 128
NPAD = 10240

NC = 2
NS = 16
CH = 64
SB = 4
NCH = 160
NCH2 = NCH // SB
EPT = NCH * CH

NACC = NPAD
RPT = NPAD // NS
ZPT = NACC // NS

BLK = 1024
NBLK = NPAD // BLK
PBLK = NPAD // BLK


def _chunks(total, step=128):
    out, o = [], 0
    while o < total:
        out.append((o, min(step, total - o)))
        o += min(step, total - o)
    return out


def _sc_agg_body(pk_hbm, hstack_hbm, out_hbm,
                 pk_v, idx_v, dst_v, rows_v, stage_v, agg_sh, sem):
    c = lax.axis_index("c")
    s = lax.axis_index("s")

    off = c * NPAD

    def zrow(r, carry):
        for k in range(HH // 16):
            stage_v[r, pl.ds(k * 16, 16)] = jnp.zeros((16,), jnp.float32)
        return carry
    lax.fori_loop(0, 128, zrow, 0)
    for (o, sz) in _chunks(ZPT):
        pltpu.sync_copy(stage_v.at[pl.ds(0, sz)],
                        agg_sh.at[pl.ds(s * ZPT + o, sz)])
    plsc.subcore_barrier()

    def sb_body(sb, carry):
        pltpu.sync_copy(pk_hbm.at[s, sb], pk_v)

        def urow(r, carry):
            for k in range(CH // 16):
                sl = pl.ds(k * 16, 16)
                v = pk_v[r, sl]
                idx_v[r, sl] = (v & 0xFFFF) + off
                dst_v[r, sl] = lax.shift_right_logical(v, 16)
            return carry
        lax.fori_loop(0, NCH2, urow, 0)

        def ebody(j, carry):
            pltpu.async_copy(hstack_hbm.at[idx_v.at[j]], rows_v, sem).wait()
            pltpu.sync_copy(rows_v, agg_sh.at[dst_v.at[j]], add=True)
            return carry
        lax.fori_loop(0, NCH2, ebody, 0)
        return carry
    lax.fori_loop(0, SB, sb_body, 0)
    plsc.subcore_barrier()

    out_base = c * NPAD + s * RPT
    for (o, sz) in _chunks(RPT):
        pltpu.sync_copy(agg_sh.at[pl.ds(s * RPT + o, sz)],
                        stage_v.at[pl.ds(0, sz)])
        pltpu.sync_copy(stage_v.at[pl.ds(0, sz)],
                        out_hbm.at[pl.ds(out_base + o, sz)])


@functools.cache
def _make_sc_agg():
    return pl.kernel(
        _sc_agg_body,
        out_type=jax.ShapeDtypeStruct((2 * NPAD, HH), jnp.float32),
        mesh=plsc.VectorSubcoreMesh(core_axis_name="c", subcore_axis_name="s",
                                    num_cores=NC, num_subcores=NS),
        scratch_types=[
            pltpu.VMEM((NCH2, CH), jnp.int32),
            pltpu.VMEM((NCH2, CH), jnp.int32),
            pltpu.VMEM((NCH2, CH), jnp.int32),
            pltpu.VMEM((CH, HH), jnp.float32),
            pltpu.VMEM((128, HH), jnp.float32),
            pltpu.VMEM_SHARED((NACC, HH), jnp.float32),
            pltpu.SemaphoreType.DMA,
        ],
    )


def _sc_agg(pk_rs, hstack):
    return _make_sc_agg()(pk_rs, hstack)


def _l0_body(x_ref, w_ref, b_ref, o_ref):
    o_ref[...] = jnp.maximum(
        jnp.dot(x_ref[...], w_ref[...], preferred_element_type=jnp.float32)
        + b_ref[0], 0.0)


def _tc_layer0(x, W0, b0):
    return pl.pallas_call(
        _l0_body,
        grid=(NC, NBLK),
        in_specs=[
            pl.BlockSpec((BLK, D), lambda c, i: (i, 0)),
            pl.BlockSpec((D, HH), lambda c, i: (0, c)),
            pl.BlockSpec((1, 1, HH), lambda c, i: (c, 0, 0)),
        ],
        out_specs=pl.BlockSpec((BLK, HH), lambda c, i: (c * PBLK + i, 0)),
        out_shape=jax.ShapeDtypeStruct((2 * NPAD, HH), jnp.float32),
    )(x, W0, b0.reshape(NC, 1, HH))


def _lmid_body(a_ref, b2_ref, w_ref, b_ref, o_ref):
    acc = jnp.dot(a_ref[...], w_ref[:HH, :], preferred_element_type=jnp.float32)
    acc += jnp.dot(b2_ref[...], w_ref[HH:, :], preferred_element_type=jnp.float32)
    o_ref[...] = jnp.maximum(acc + b_ref[0], 0.0)


def _tc_layer(agg, W, b):
    return pl.pallas_call(
        _lmid_body,
        grid=(NC, NBLK),
        in_specs=[
            pl.BlockSpec((BLK, HH), lambda c, i: (i, 0)),
            pl.BlockSpec((BLK, HH), lambda c, i: (PBLK + i, 0)),
            pl.BlockSpec((H, HH), lambda c, i: (0, c)),
            pl.BlockSpec((1, 1, HH), lambda c, i: (c, 0, 0)),
        ],
        out_specs=pl.BlockSpec((BLK, HH), lambda c, i: (c * PBLK + i, 0)),
        out_shape=jax.ShapeDtypeStruct((2 * NPAD, HH), jnp.float32),
    )(agg, agg, W, b.reshape(NC, 1, HH))


def _final_body(a_ref, b2_ref, w_ref, b_ref, wp_ref, bp_ref, o_ref):
    acc = jnp.dot(a_ref[...], w_ref[:HH, :], preferred_element_type=jnp.float32)
    acc += jnp.dot(b2_ref[...], w_ref[HH:, :], preferred_element_type=jnp.float32)
    h = jnp.maximum(acc + b_ref[...], 0.0)
    o_ref[...] = jnp.dot(h, wp_ref[...], preferred_element_type=jnp.float32) \
        + bp_ref[...]


def _tc_final(agg, W, b, Wp, bp):
    return pl.pallas_call(
        _final_body,
        grid=(NBLK,),
        in_specs=[
            pl.BlockSpec((BLK, HH), lambda i: (i, 0)),
            pl.BlockSpec((BLK, HH), lambda i: (PBLK + i, 0)),
            pl.BlockSpec((H, H), lambda i: (0, 0)),
            pl.BlockSpec((1, H), lambda i: (0, 0)),
            pl.BlockSpec((H, 1), lambda i: (0, 0)),
            pl.BlockSpec((1, 1), lambda i: (0, 0)),
        ],
        out_specs=pl.BlockSpec((BLK, 1), lambda i: (i, 0)),
        out_shape=jax.ShapeDtypeStruct((NPAD, 1), jnp.float32),
    )(agg, agg, W, b.reshape(1, H), Wp, bp.reshape(1, 1))


def kernel(x, edge_index, W0, b0, W1, b1, W2, b2, W3, b3, W4, b4, Wp, bp):
    src = edge_index[0]
    dst = edge_index[1]
    pad = EPT - E // NS
    pk = (dst << 16) | src
    pk_rs = jnp.pad(pk.reshape(NS, E // NS), ((0, 0), (0, pad)),
                    constant_values=N << 16).reshape(NS, SB, NCH2, CH)

    x_pad = jnp.pad(x, ((0, NPAD - N), (0, 0)))
    h = _tc_layer0(x_pad, W0, b0)
    for W, b in ((W1, b1), (W2, b2), (W3, b3)):
        agg = _sc_agg(pk_rs, h)
        h = _tc_layer(agg, W, b)
    agg = _sc_agg(pk_rs, h)
    return _tc_final(agg, W4, b4, Wp, bp)[:N]


# speedup vs baseline: 2.2862x; 1.0584x over previous
import functools

import jax
import jax.numpy as jnp
from jax import lax
from jax.experimental import pallas as pl
from jax.experimental.pallas import tpu as pltpu
from jax.experimental.pallas import tpu_sc as plsc

N = 10000
E = 160000
D = 256
H = 256
HH = 128
NPAD = 10240

NC = 2
NS = 16
CH = 80
SB = 4
NCH = 128
NCH2 = NCH // SB
EPT = NCH * CH

NACC = NPAD
RPT = NPAD // NS
ZPT = NACC // NS

BLK = 1024
NBLK = NPAD // BLK
PBLK = NPAD // BLK


def _chunks(total, step=128):
    out, o = [], 0
    while o < total:
        out.append((o, min(step, total - o)))
        o += min(step, total - o)
    return out


def _sc_agg_body(pk_hbm, hstack_hbm, out_hbm,
                 pk_v, idx_v, dst_v, rows_v, stage_v, agg_sh, sem):
    c = lax.axis_index("c")
    s = lax.axis_index("s")

    off = c * NPAD

    def zrow(r, carry):
        for k in range(HH // 16):
            stage_v[r, pl.ds(k * 16, 16)] = jnp.zeros((16,), jnp.float32)
        return carry
    lax.fori_loop(0, 128, zrow, 0)
    for (o, sz) in _chunks(ZPT):
        pltpu.sync_copy(stage_v.at[pl.ds(0, sz)],
                        agg_sh.at[pl.ds(s * ZPT + o, sz)])
    plsc.subcore_barrier()

    def sb_body(sb, carry):
        pltpu.sync_copy(pk_hbm.at[s, sb], pk_v)

        def urow(r, carry):
            for k in range(CH // 16):
                sl = pl.ds(k * 16, 16)
                v = pk_v[r, sl]
                idx_v[r, sl] = (v & 0xFFFF) + off
                dst_v[r, sl] = lax.shift_right_logical(v, 16)
            return carry
        lax.fori_loop(0, NCH2, urow, 0)

        def ebody(j, carry):
            pltpu.async_copy(hstack_hbm.at[idx_v.at[j]], rows_v, sem).wait()
            pltpu.sync_copy(rows_v, agg_sh.at[dst_v.at[j]], add=True)
            return carry
        lax.fori_loop(0, NCH2, ebody, 0)
        return carry
    lax.fori_loop(0, SB, sb_body, 0)
    plsc.subcore_barrier()

    out_base = c * NPAD + s * RPT
    for (o, sz) in _chunks(RPT):
        pltpu.sync_copy(agg_sh.at[pl.ds(s * RPT + o, sz)],
                        stage_v.at[pl.ds(0, sz)])
        pltpu.sync_copy(stage_v.at[pl.ds(0, sz)],
                        out_hbm.at[pl.ds(out_base + o, sz)])


@functools.cache
def _make_sc_agg():
    return pl.kernel(
        _sc_agg_body,
        out_type=jax.ShapeDtypeStruct((2 * NPAD, HH), jnp.float32),
        mesh=plsc.VectorSubcoreMesh(core_axis_name="c", subcore_axis_name="s",
                                    num_cores=NC, num_subcores=NS),
        scratch_types=[
            pltpu.VMEM((NCH2, CH), jnp.int32),
            pltpu.VMEM((NCH2, CH), jnp.int32),
            pltpu.VMEM((NCH2, CH), jnp.int32),
            pltpu.VMEM((CH, HH), jnp.float32),
            pltpu.VMEM((128, HH), jnp.float32),
            pltpu.VMEM_SHARED((NACC, HH), jnp.float32),
            pltpu.SemaphoreType.DMA,
        ],
    )


def _sc_agg(pk_rs, hstack):
    return _make_sc_agg()(pk_rs, hstack)


def _l0_body(x_ref, w_ref, b_ref, o_ref):
    o_ref[...] = jnp.maximum(
        jnp.dot(x_ref[...], w_ref[...], preferred_element_type=jnp.float32)
        + b_ref[0], 0.0)


def _tc_layer0(x, W0, b0):
    return pl.pallas_call(
        _l0_body,
        grid=(NC, NBLK),
        in_specs=[
            pl.BlockSpec((BLK, D), lambda c, i: (i, 0)),
            pl.BlockSpec((D, HH), lambda c, i: (0, c)),
            pl.BlockSpec((1, 1, HH), lambda c, i: (c, 0, 0)),
        ],
        out_specs=pl.BlockSpec((BLK, HH), lambda c, i: (c * PBLK + i, 0)),
        out_shape=jax.ShapeDtypeStruct((2 * NPAD, HH), jnp.float32),
    )(x, W0, b0.reshape(NC, 1, HH))


def _lmid_body(a_ref, b2_ref, w_ref, b_ref, o_ref):
    acc = jnp.dot(a_ref[...], w_ref[:HH, :], preferred_element_type=jnp.float32)
    acc += jnp.dot(b2_ref[...], w_ref[HH:, :], preferred_element_type=jnp.float32)
    o_ref[...] = jnp.maximum(acc + b_ref[0], 0.0)


def _tc_layer(agg, W, b):
    return pl.pallas_call(
        _lmid_body,
        grid=(NC, NBLK),
        in_specs=[
            pl.BlockSpec((BLK, HH), lambda c, i: (i, 0)),
            pl.BlockSpec((BLK, HH), lambda c, i: (PBLK + i, 0)),
            pl.BlockSpec((H, HH), lambda c, i: (0, c)),
            pl.BlockSpec((1, 1, HH), lambda c, i: (c, 0, 0)),
        ],
        out_specs=pl.BlockSpec((BLK, HH), lambda c, i: (c * PBLK + i, 0)),
        out_shape=jax.ShapeDtypeStruct((2 * NPAD, HH), jnp.float32),
    )(agg, agg, W, b.reshape(NC, 1, HH))


def _final_body(a_ref, b2_ref, w_ref, b_ref, wp_ref, bp_ref, o_ref):
    acc = jnp.dot(a_ref[...], w_ref[:HH, :], preferred_element_type=jnp.float32)
    acc += jnp.dot(b2_ref[...], w_ref[HH:, :], preferred_element_type=jnp.float32)
    h = jnp.maximum(acc + b_ref[...], 0.0)
    o_ref[...] = jnp.dot(h, wp_ref[...], preferred_element_type=jnp.float32) \
        + bp_ref[...]


def _tc_final(agg, W, b, Wp, bp):
    return pl.pallas_call(
        _final_body,
        grid=(NBLK,),
        in_specs=[
            pl.BlockSpec((BLK, HH), lambda i: (i, 0)),
            pl.BlockSpec((BLK, HH), lambda i: (PBLK + i, 0)),
            pl.BlockSpec((H, H), lambda i: (0, 0)),
            pl.BlockSpec((1, H), lambda i: (0, 0)),
            pl.BlockSpec((H, 1), lambda i: (0, 0)),
            pl.BlockSpec((1, 1), lambda i: (0, 0)),
        ],
        out_specs=pl.BlockSpec((BLK, 1), lambda i: (i, 0)),
        out_shape=jax.ShapeDtypeStruct((NPAD, 1), jnp.float32),
    )(agg, agg, W, b.reshape(1, H), Wp, bp.reshape(1, 1))


def kernel(x, edge_index, W0, b0, W1, b1, W2, b2, W3, b3, W4, b4, Wp, bp):
    src = edge_index[0]
    dst = edge_index[1]
    pad = EPT - E // NS
    pk = (dst << 16) | src
    pk_rs = jnp.pad(pk.reshape(NS, E // NS), ((0, 0), (0, pad)),
                    constant_values=N << 16).reshape(NS, SB, NCH2, CH)

    x_pad = jnp.pad(x, ((0, NPAD - N), (0, 0)))
    h = _tc_layer0(x_pad, W0, b0)
    for W, b in ((W1, b1), (W2, b2), (W3, b3)):
        agg = _sc_agg(pk_rs, h)
        h = _tc_layer(agg, W, b)
    agg = _sc_agg(pk_rs, h)
    return _tc_final(agg, W4, b4, Wp, bp)[:N]


# speedup vs baseline: 2.4739x; 1.0821x over previous
import functools

import jax
import jax.numpy as jnp
from jax import lax
from jax.experimental import pallas as pl
from jax.experimental.pallas import tpu as pltpu
from jax.experimental.pallas import tpu_sc as plsc

N = 10000
E = 160000
D = 256
H = 256
HH = 128
NPAD = 10240

NC = 2
NS = 16
CH = 128
SB = 8
NCH = 80
NCH2 = NCH // SB
EPT = NCH * CH

NACC = NPAD
RPT = NPAD // NS
ZPT = NACC // NS

BLK = 1024
NBLK = NPAD // BLK
PBLK = NPAD // BLK


def _chunks(total, step=128):
    out, o = [], 0
    while o < total:
        out.append((o, min(step, total - o)))
        o += min(step, total - o)
    return out


def _sc_agg_body(pk_hbm, hstack_hbm, out_hbm,
                 pk_v, idx_v, dst_v, rows_v, stage_v, agg_sh, sem):
    c = lax.axis_index("c")
    s = lax.axis_index("s")

    off = c * NPAD

    def zrow(r, carry):
        for k in range(HH // 16):
            stage_v[r, pl.ds(k * 16, 16)] = jnp.zeros((16,), jnp.float32)
        return carry
    lax.fori_loop(0, 128, zrow, 0)
    for (o, sz) in _chunks(ZPT):
        pltpu.sync_copy(stage_v.at[pl.ds(0, sz)],
                        agg_sh.at[pl.ds(s * ZPT + o, sz)])
    plsc.subcore_barrier()

    def sb_body(sb, carry):
        pltpu.sync_copy(pk_hbm.at[s, sb], pk_v)

        def urow(r, carry):
            for k in range(CH // 16):
                sl = pl.ds(k * 16, 16)
                v = pk_v[r, sl]
                idx_v[r, sl] = (v & 0xFFFF) + off
                dst_v[r, sl] = lax.shift_right_logical(v, 16)
            return carry
        lax.fori_loop(0, NCH2, urow, 0)

        def ebody(j, carry):
            pltpu.async_copy(hstack_hbm.at[idx_v.at[j]], rows_v, sem).wait()
            pltpu.sync_copy(rows_v, agg_sh.at[dst_v.at[j]], add=True)
            return carry
        lax.fori_loop(0, NCH2, ebody, 0)
        return carry
    lax.fori_loop(0, SB, sb_body, 0)
    plsc.subcore_barrier()

    out_base = c * NPAD + s * RPT
    for (o, sz) in _chunks(RPT):
        pltpu.sync_copy(agg_sh.at[pl.ds(s * RPT + o, sz)],
                        stage_v.at[pl.ds(0, sz)])
        pltpu.sync_copy(stage_v.at[pl.ds(0, sz)],
                        out_hbm.at[pl.ds(out_base + o, sz)])


@functools.cache
def _make_sc_agg():
    return pl.kernel(
        _sc_agg_body,
        out_type=jax.ShapeDtypeStruct((2 * NPAD, HH), jnp.float32),
        mesh=plsc.VectorSubcoreMesh(core_axis_name="c", subcore_axis_name="s",
                                    num_cores=NC, num_subcores=NS),
        scratch_types=[
            pltpu.VMEM((NCH2, CH), jnp.int32),
            pltpu.VMEM((NCH2, CH), jnp.int32),
            pltpu.VMEM((NCH2, CH), jnp.int32),
            pltpu.VMEM((CH, HH), jnp.float32),
            pltpu.VMEM((128, HH), jnp.float32),
            pltpu.VMEM_SHARED((NACC, HH), jnp.float32),
            pltpu.SemaphoreType.DMA,
        ],
    )


def _sc_agg(pk_rs, hstack):
    return _make_sc_agg()(pk_rs, hstack)


def _l0_body(x_ref, w_ref, b_ref, o_ref):
    o_ref[...] = jnp.maximum(
        jnp.dot(x_ref[...], w_ref[...], preferred_element_type=jnp.float32)
        + b_ref[0], 0.0)


def _tc_layer0(x, W0, b0):
    return pl.pallas_call(
        _l0_body,
        grid=(NC, NBLK),
        in_specs=[
            pl.BlockSpec((BLK, D), lambda c, i: (i, 0)),
            pl.BlockSpec((D, HH), lambda c, i: (0, c)),
            pl.BlockSpec((1, 1, HH), lambda c, i: (c, 0, 0)),
        ],
        out_specs=pl.BlockSpec((BLK, HH), lambda c, i: (c * PBLK + i, 0)),
        out_shape=jax.ShapeDtypeStruct((2 * NPAD, HH), jnp.float32),
    )(x, W0, b0.reshape(NC, 1, HH))


def _lmid_body(a_ref, b2_ref, w_ref, b_ref, o_ref):
    acc = jnp.dot(a_ref[...], w_ref[:HH, :], preferred_element_type=jnp.float32)
    acc += jnp.dot(b2_ref[...], w_ref[HH:, :], preferred_element_type=jnp.float32)
    o_ref[...] = jnp.maximum(acc + b_ref[0], 0.0)


def _tc_layer(agg, W, b):
    return pl.pallas_call(
        _lmid_body,
        grid=(NC, NBLK),
        in_specs=[
            pl.BlockSpec((BLK, HH), lambda c, i: (i, 0)),
            pl.BlockSpec((BLK, HH), lambda c, i: (PBLK + i, 0)),
            pl.BlockSpec((H, HH), lambda c, i: (0, c)),
            pl.BlockSpec((1, 1, HH), lambda c, i: (c, 0, 0)),
        ],
        out_specs=pl.BlockSpec((BLK, HH), lambda c, i: (c * PBLK + i, 0)),
        out_shape=jax.ShapeDtypeStruct((2 * NPAD, HH), jnp.float32),
    )(agg, agg, W, b.reshape(NC, 1, HH))


def _final_body(a_ref, b2_ref, w_ref, b_ref, wp_ref, bp_ref, o_ref):
    acc = jnp.dot(a_ref[...], w_ref[:HH, :], preferred_element_type=jnp.float32)
    acc += jnp.dot(b2_ref[...], w_ref[HH:, :], preferred_element_type=jnp.float32)
    h = jnp.maximum(acc + b_ref[...], 0.0)
    o_ref[...] = jnp.dot(h, wp_ref[...], preferred_element_type=jnp.float32) \
        + bp_ref[...]


def _tc_final(agg, W, b, Wp, bp):
    return pl.pallas_call(
        _final_body,
        grid=(NBLK,),
        in_specs=[
            pl.BlockSpec((BLK, HH), lambda i: (i, 0)),
            pl.BlockSpec((BLK, HH), lambda i: (PBLK + i, 0)),
            pl.BlockSpec((H, H), lambda i: (0, 0)),
            pl.BlockSpec((1, H), lambda i: (0, 0)),
            pl.BlockSpec((H, 1), lambda i: (0, 0)),
            pl.BlockSpec((1, 1), lambda i: (0, 0)),
        ],
        out_specs=pl.BlockSpec((BLK, 1), lambda i: (i, 0)),
        out_shape=jax.ShapeDtypeStruct((NPAD, 1), jnp.float32),
    )(agg, agg, W, b.reshape(1, H), Wp, bp.reshape(1, 1))


def kernel(x, edge_index, W0, b0, W1, b1, W2, b2, W3, b3, W4, b4, Wp, bp):
    src = edge_index[0]
    dst = edge_index[1]
    pad = EPT - E // NS
    pk = (dst << 16) | src
    pk_rs = jnp.pad(pk.reshape(NS, E // NS), ((0, 0), (0, pad)),
                    constant_values=N << 16).reshape(NS, SB, NCH2, CH)

    x_pad = jnp.pad(x, ((0, NPAD - N), (0, 0)))
    h = _tc_layer0(x_pad, W0, b0)
    for W, b in ((W1, b1), (W2, b2), (W3, b3)):
        agg = _sc_agg(pk_rs, h)
        h = _tc_layer(agg, W, b)
    agg = _sc_agg(pk_rs, h)
    return _tc_final(agg, W4, b4, Wp, bp)[:N]


# speedup vs baseline: 2.8671x; 1.1590x over previous
import functools

import jax
import jax.numpy as jnp
from jax import lax
from jax.experimental import pallas as pl
from jax.experimental.pallas import tpu as pltpu
from jax.experimental.pallas import tpu_sc as plsc

N = 10000
E = 160000
D = 256
H = 256
HH = 128
NPAD = 10240

NC = 2
NS = 16
CH = 80
SB = 8
NCH = 128
NCH2 = NCH // SB
EPT = NCH * CH

NACC = NPAD
RPT = NPAD // NS
ZPT = NACC // NS

BLK = 1024
NBLK = NPAD // BLK
PBLK = NPAD // BLK


def _chunks(total, step=128):
    out, o = [], 0
    while o < total:
        out.append((o, min(step, total - o)))
        o += min(step, total - o)
    return out


def _sc_agg_body(pk_hbm, hstack_hbm, out_hbm,
                 pk_v, idx_v, dst_v, rows_v, rows2_v, stage_v, agg_sh,
                 sem, sem2):
    c = lax.axis_index("c")
    s = lax.axis_index("s")

    off = c * NPAD

    def zrow(r, carry):
        for k in range(HH // 16):
            stage_v[r, pl.ds(k * 16, 16)] = jnp.zeros((16,), jnp.float32)
        return carry
    lax.fori_loop(0, 128, zrow, 0)
    for (o, sz) in _chunks(ZPT):
        pltpu.sync_copy(stage_v.at[pl.ds(0, sz)],
                        agg_sh.at[pl.ds(s * ZPT + o, sz)])
    plsc.subcore_barrier()

    def sb_body(sb, carry):
        pltpu.sync_copy(pk_hbm.at[s, sb], pk_v)

        def urow(r, carry):
            for k in range(CH // 16):
                sl = pl.ds(k * 16, 16)
                v = pk_v[r, sl]
                idx_v[r, sl] = (v & 0xFFFF) + off
                dst_v[r, sl] = lax.shift_right_logical(v, 16)
            return carry
        lax.fori_loop(0, NCH2, urow, 0)

        bufs = (rows_v, rows2_v)
        sems = (sem, sem2)
        hs = [None, None]
        hs[0] = pltpu.async_copy(hstack_hbm.at[idx_v.at[0]], rows_v, sem)
        for j in range(NCH2):
            b = j % 2
            nb = (j + 1) % 2
            if j + 1 < NCH2:
                hs[nb] = pltpu.async_copy(
                    hstack_hbm.at[idx_v.at[j + 1]], bufs[nb], sems[nb])
            hs[b].wait()
            pltpu.sync_copy(bufs[b], agg_sh.at[dst_v.at[j]], add=True)
        return carry
    lax.fori_loop(0, SB, sb_body, 0)
    plsc.subcore_barrier()

    out_base = c * NPAD + s * RPT
    for (o, sz) in _chunks(RPT):
        pltpu.sync_copy(agg_sh.at[pl.ds(s * RPT + o, sz)],
                        stage_v.at[pl.ds(0, sz)])
        pltpu.sync_copy(stage_v.at[pl.ds(0, sz)],
                        out_hbm.at[pl.ds(out_base + o, sz)])


@functools.cache
def _make_sc_agg():
    return pl.kernel(
        _sc_agg_body,
        out_type=jax.ShapeDtypeStruct((2 * NPAD, HH), jnp.float32),
        mesh=plsc.VectorSubcoreMesh(core_axis_name="c", subcore_axis_name="s",
                                    num_cores=NC, num_subcores=NS),
        scratch_types=[
            pltpu.VMEM((NCH2, CH), jnp.int32),
            pltpu.VMEM((NCH2, CH), jnp.int32),
            pltpu.VMEM((NCH2, CH), jnp.int32),
            pltpu.VMEM((CH, HH), jnp.float32),
            pltpu.VMEM((CH, HH), jnp.float32),
            pltpu.VMEM((128, HH), jnp.float32),
            pltpu.VMEM_SHARED((NACC, HH), jnp.float32),
            pltpu.SemaphoreType.DMA,
            pltpu.SemaphoreType.DMA,
        ],
    )


def _sc_agg(pk_rs, hstack):
    return _make_sc_agg()(pk_rs, hstack)


def _l0_body(x_ref, w_ref, b_ref, o_ref):
    o_ref[...] = jnp.maximum(
        jnp.dot(x_ref[...], w_ref[...], preferred_element_type=jnp.float32)
        + b_ref[0], 0.0)


def _tc_layer0(x, W0, b0):
    return pl.pallas_call(
        _l0_body,
        grid=(NC, NBLK),
        in_specs=[
            pl.BlockSpec((BLK, D), lambda c, i: (i, 0)),
            pl.BlockSpec((D, HH), lambda c, i: (0, c)),
            pl.BlockSpec((1, 1, HH), lambda c, i: (c, 0, 0)),
        ],
        out_specs=pl.BlockSpec((BLK, HH), lambda c, i: (c * PBLK + i, 0)),
        out_shape=jax.ShapeDtypeStruct((2 * NPAD, HH), jnp.float32),
    )(x, W0, b0.reshape(NC, 1, HH))


def _lmid_body(a_ref, b2_ref, w_ref, b_ref, o_ref):
    acc = jnp.dot(a_ref[...], w_ref[:HH, :], preferred_element_type=jnp.float32)
    acc += jnp.dot(b2_ref[...], w_ref[HH:, :], preferred_element_type=jnp.float32)
    o_ref[...] = jnp.maximum(acc + b_ref[0], 0.0)


def _tc_layer(agg, W, b):
    return pl.pallas_call(
        _lmid_body,
        grid=(NC, NBLK),
        in_specs=[
            pl.BlockSpec((BLK, HH), lambda c, i: (i, 0)),
            pl.BlockSpec((BLK, HH), lambda c, i: (PBLK + i, 0)),
            pl.BlockSpec((H, HH), lambda c, i: (0, c)),
            pl.BlockSpec((1, 1, HH), lambda c, i: (c, 0, 0)),
        ],
        out_specs=pl.BlockSpec((BLK, HH), lambda c, i: (c * PBLK + i, 0)),
        out_shape=jax.ShapeDtypeStruct((2 * NPAD, HH), jnp.float32),
    )(agg, agg, W, b.reshape(NC, 1, HH))


def _final_body(a_ref, b2_ref, w_ref, b_ref, wp_ref, bp_ref, o_ref):
    acc = jnp.dot(a_ref[...], w_ref[:HH, :], preferred_element_type=jnp.float32)
    acc += jnp.dot(b2_ref[...], w_ref[HH:, :], preferred_element_type=jnp.float32)
    h = jnp.maximum(acc + b_ref[...], 0.0)
    o_ref[...] = jnp.dot(h, wp_ref[...], preferred_element_type=jnp.float32) \
        + bp_ref[...]


def _tc_final(agg, W, b, Wp, bp):
    return pl.pallas_call(
        _final_body,
        grid=(NBLK,),
        in_specs=[
            pl.BlockSpec((BLK, HH), lambda i: (i, 0)),
            pl.BlockSpec((BLK, HH), lambda i: (PBLK + i, 0)),
            pl.BlockSpec((H, H), lambda i: (0, 0)),
            pl.BlockSpec((1, H), lambda i: (0, 0)),
            pl.BlockSpec((H, 1), lambda i: (0, 0)),
            pl.BlockSpec((1, 1), lambda i: (0, 0)),
        ],
        out_specs=pl.BlockSpec((BLK, 1), lambda i: (i, 0)),
        out_shape=jax.ShapeDtypeStruct((NPAD, 1), jnp.float32),
    )(agg, agg, W, b.reshape(1, H), Wp, bp.reshape(1, 1))


def kernel(x, edge_index, W0, b0, W1, b1, W2, b2, W3, b3, W4, b4, Wp, bp):
    src = edge_index[0]
    dst = edge_index[1]
    pad = EPT - E // NS
    pk = (dst << 16) | src
    pk_rs = jnp.pad(pk.reshape(NS, E // NS), ((0, 0), (0, pad)),
                    constant_values=N << 16).reshape(NS, SB, NCH2, CH)

    x_pad = jnp.pad(x, ((0, NPAD - N), (0, 0)))
    h = _tc_layer0(x_pad, W0, b0)
    for W, b in ((W1, b1), (W2, b2), (W3, b3)):
        agg = _sc_agg(pk_rs, h)
        h = _tc_layer(agg, W, b)
    agg = _sc_agg(pk_rs, h)
    return _tc_final(agg, W4, b4, Wp, bp)[:N]
